# baseline (speedup 1.0000x reference)
import functools

import jax
import jax.numpy as jnp
from jax import lax
from jax.experimental import pallas as pl
from jax.experimental.pallas import tpu as pltpu
from jax.experimental.pallas import tpu_sc as plsc

H = 1024
F = 2048
E = 8
KSEL = 2
N = 2048
M = N * KSEL

T = 128
NT = M // T + E
S = NT * T


def _router_body(x_ref, wr_ref, ee_ref, ww_ref):
    x = x_ref[...]
    wr = wr_ref[...]
    logits = jax.lax.dot_general(
        x, wr, (((1,), (1,)), ((), ())), preferred_element_type=jnp.float32)
    mx = jnp.max(logits, axis=-1, keepdims=True)
    ex = jnp.exp(logits - mx)
    p = ex / jnp.sum(ex, axis=-1, keepdims=True)
    lane = jax.lax.broadcasted_iota(jnp.int32, (N, E), 1)
    m1 = jnp.max(p, axis=-1, keepdims=True)
    i1 = jnp.min(jnp.where(p == m1, lane, E), axis=-1, keepdims=True)
    p2 = jnp.where(lane == i1, -jnp.inf, p)
    m2 = jnp.max(p2, axis=-1, keepdims=True)
    i2 = jnp.min(jnp.where(p2 == m2, lane, E), axis=-1, keepdims=True)
    s = m1 + m2
    ee_ref[...] = jnp.where(lane % 2 == 0, i1, i2)
    ww_ref[...] = jnp.where(lane % 2 == 0, m1 / s, m2 / s)


def _router(x, Wr):
    ee, ww = pl.pallas_call(
        _router_body,
        out_shape=(jax.ShapeDtypeStruct((N, E), jnp.int32),
                   jax.ShapeDtypeStruct((N, E), jnp.float32)),
    )(x, Wr)
    return ee[:, 0], ee[:, 1], ww[:, 0], ww[:, 1]


def _mlp_body(meta_ref, xs_ref, wg_ref, wu_ref, wd_ref, os_ref):
    t = pl.program_id(0)

    @pl.when(t < meta_ref[NT])
    def _():
        xv = xs_ref[...].astype(jnp.bfloat16)
        g = jax.lax.dot_general(
            xv, wg_ref[0].astype(jnp.bfloat16), (((1,), (1,)), ((), ())),
            preferred_element_type=jnp.float32)
        u = jax.lax.dot_general(
            xv, wu_ref[0].astype(jnp.bfloat16), (((1,), (1,)), ((), ())),
            preferred_element_type=jnp.float32)
        a = g * jax.nn.sigmoid(g) * u
        os_ref[...] = jax.lax.dot_general(
            a.astype(jnp.bfloat16), wd_ref[0].astype(jnp.bfloat16),
            (((1,), (1,)), ((), ())),
            preferred_element_type=jnp.float32)


def _grouped_mlp(meta, xs, Wg, Wu, Wd):
    grid_spec = pltpu.PrefetchScalarGridSpec(
        num_scalar_prefetch=1,
        grid=(NT,),
        in_specs=[
            pl.BlockSpec((T, H), lambda t, m: (t, 0)),
            pl.BlockSpec((1, F, H), lambda t, m: (m[t], 0, 0)),
            pl.BlockSpec((1, F, H), lambda t, m: (m[t], 0, 0)),
            pl.BlockSpec((1, H, F), lambda t, m: (m[t], 0, 0)),
        ],
        out_specs=pl.BlockSpec((T, H), lambda t, m: (t, 0)),
    )
    return pl.pallas_call(
        _mlp_body,
        grid_spec=grid_spec,
        out_shape=jax.ShapeDtypeStruct((S, H), jnp.float32),
        compiler_params=pltpu.CompilerParams(
            dimension_semantics=("arbitrary",)),
    )(meta, xs, Wg, Wu, Wd)


_SC_MESH = plsc.VectorSubcoreMesh(core_axis_name="c", subcore_axis_name="s")
_NW = 32
_TPW = N // _NW


def _dispatch_body(x_hbm, s1_hbm, s2_hbm, xs_hbm, rows_v, i1_v, i2_v, sem):
    wid = lax.axis_index("s") * 2 + lax.axis_index("c")
    tok0 = wid * _TPW
    pltpu.sync_copy(s1_hbm.at[pl.ds(tok0, _TPW)], i1_v)
    pltpu.sync_copy(s2_hbm.at[pl.ds(tok0, _TPW)], i2_v)
    pltpu.sync_copy(x_hbm.at[pl.ds(tok0, _TPW)], rows_v)
    c1 = pltpu.async_copy(rows_v, xs_hbm.at[i1_v], sem)
    c2 = pltpu.async_copy(rows_v, xs_hbm.at[i2_v], sem)
    c1.wait()
    c2.wait()


def _sc_dispatch(x, slot1, slot2):
    return pl.kernel(
        _dispatch_body,
        mesh=_SC_MESH,
        out_type=jax.ShapeDtypeStruct((S, H), jnp.float32),
        scratch_types=[
            pltpu.VMEM((_TPW, H), jnp.float32),
            pltpu.VMEM((_TPW,), jnp.int32),
            pltpu.VMEM((_TPW,), jnp.int32),
            pltpu.SemaphoreType.DMA,
        ],
        compiler_params=pltpu.CompilerParams(needs_layout_passes=False),
    )(x, slot1, slot2)


_CSUB = 32


def _combine_body(os_hbm, s1_hbm, s2_hbm, w1_hbm, w2_hbm, out_hbm,
                  r1_v, r2_v, i1_v, i2_v, w1_v, w2_v, sem):
    wid = lax.axis_index("s") * 2 + lax.axis_index("c")
    for sub in range(_TPW // _CSUB):
        tok0 = wid * _TPW + sub * _CSUB
        pltpu.sync_copy(s1_hbm.at[pl.ds(tok0, _CSUB)], i1_v)
        pltpu.sync_copy(s2_hbm.at[pl.ds(tok0, _CSUB)], i2_v)
        pltpu.sync_copy(w1_hbm.at[pl.ds(tok0, _CSUB)], w1_v)
        pltpu.sync_copy(w2_hbm.at[pl.ds(tok0, _CSUB)], w2_v)
        c1 = pltpu.async_copy(os_hbm.at[i1_v], r1_v, sem)
        c2 = pltpu.async_copy(os_hbm.at[i2_v], r2_v, sem)
        c1.wait()
        c2.wait()

        def tok_body(i, _):
            w1b = plsc.load_gather(w1_v, [jnp.full((16,), i, jnp.int32)])
            w2b = plsc.load_gather(w2_v, [jnp.full((16,), i, jnp.int32)])
            for j in range(H // 16):
                sl = pl.ds(j * 16, 16)
                r1_v[i, sl] = w1b * r1_v[i, sl] + w2b * r2_v[i, sl]
            return 0

        lax.fori_loop(0, _CSUB, tok_body, 0)
        pltpu.sync_copy(r1_v, out_hbm.at[pl.ds(tok0, _CSUB)])


def _sc_combine(os_arr, slot1, slot2, w1, w2):
    return pl.kernel(
        _combine_body,
        mesh=_SC_MESH,
        out_type=jax.ShapeDtypeStruct((N, H), jnp.float32),
        scratch_types=[
            pltpu.VMEM((_CSUB, H), jnp.float32),
            pltpu.VMEM((_CSUB, H), jnp.float32),
            pltpu.VMEM((_CSUB,), jnp.int32),
            pltpu.VMEM((_CSUB,), jnp.int32),
            pltpu.VMEM((_CSUB,), jnp.float32),
            pltpu.VMEM((_CSUB,), jnp.float32),
            pltpu.SemaphoreType.DMA,
        ],
        compiler_params=pltpu.CompilerParams(needs_layout_passes=False),
    )(os_arr, slot1, slot2, w1, w2)


def kernel(x, Wr, Wg, Wu, Wd):
    orig_shape = x.shape
    xf = x.reshape(-1, H)
    e1, e2, w1, w2 = _router(xf, Wr)

    e = jnp.stack([e1, e2], axis=1).reshape(-1)
    oh = (e[:, None] == jnp.arange(E)[None, :]).astype(jnp.int32)
    rank = jnp.cumsum(oh, axis=0) - oh
    counts = jnp.sum(oh, axis=0)
    tiles_per = (counts + T - 1) // T
    padded = tiles_per * T
    start = jnp.cumsum(padded) - padded
    pair_slot = start[e] + jnp.sum(rank * oh, axis=1)
    cum_tiles = jnp.cumsum(tiles_per)
    used = cum_tiles[-1]
    tt = jnp.arange(NT, dtype=jnp.int32)
    tile_expert = jnp.sum(
        (tt[:, None] >= cum_tiles[None, :]).astype(jnp.int32), axis=1)
    tile_expert = jnp.minimum(tile_expert, E - 1)
    meta = jnp.concatenate(
        [tile_expert, used[None]]).astype(jnp.int32)

    slot = pair_slot.reshape(N, KSEL)
    slot1, slot2 = slot[:, 0], slot[:, 1]
    xs = _sc_dispatch(xf, slot1, slot2)

    os_arr = _grouped_mlp(meta, xs, Wg, Wu, Wd)

    final = _sc_combine(os_arr, slot1, slot2, w1, w2)
    return final.reshape(orig_shape)


# speedup vs baseline: 1.4875x; 1.4875x over previous
import functools

import jax
import jax.numpy as jnp
from jax import lax
from jax.experimental import pallas as pl
from jax.experimental.pallas import tpu as pltpu
from jax.experimental.pallas import tpu_sc as plsc

H = 1024
F = 2048
E = 8
KSEL = 2
N = 2048
M = N * KSEL

T = 256
NT = M // T + E
S = NT * T


def _router_body(x_ref, wr_ref, ss_ref, ww_ref, meta_ref):
    x = x_ref[...]
    wr = wr_ref[...]
    logits = jax.lax.dot_general(
        x, wr, (((1,), (1,)), ((), ())), preferred_element_type=jnp.float32)
    mx = jnp.max(logits, axis=-1, keepdims=True)
    ex = jnp.exp(logits - mx)
    p = ex / jnp.sum(ex, axis=-1, keepdims=True)
    lane = jax.lax.broadcasted_iota(jnp.int32, (N, E), 1)
    m1 = jnp.max(p, axis=-1, keepdims=True)
    i1 = jnp.min(jnp.where(p == m1, lane, E), axis=-1, keepdims=True)
    p2 = jnp.where(lane == i1, -jnp.inf, p)
    m2 = jnp.max(p2, axis=-1, keepdims=True)
    i2 = jnp.min(jnp.where(p2 == m2, lane, E), axis=-1, keepdims=True)
    s = m1 + m2
    ww_ref[...] = jnp.where(lane % 2 == 0, m1 / s, m2 / s)

    sel1 = (lane == i1)
    sel2 = (lane == i2)
    ohf = (sel1 | sel2).astype(jnp.bfloat16)
    rio = jax.lax.broadcasted_iota(jnp.int32, (N, N), 0)
    cio = jax.lax.broadcasted_iota(jnp.int32, (N, N), 1)
    ltri = (cio < rio).astype(jnp.bfloat16)
    rankb = jax.lax.dot_general(
        ltri, ohf, (((1,), (0,)), ((), ())),
        preferred_element_type=jnp.float32)
    counts = jnp.sum(ohf.astype(jnp.float32), axis=0, keepdims=True)
    tiles_per = (counts.astype(jnp.int32) + (T - 1)) // T
    padded = (tiles_per * T).astype(jnp.float32)
    erow = jax.lax.broadcasted_iota(jnp.int32, (E, E), 0)
    ecol = jax.lax.broadcasted_iota(jnp.int32, (E, E), 1)
    g_lt = (erow < ecol).astype(jnp.float32)
    start = jax.lax.dot_general(
        padded, g_lt, (((1,), (0,)), ((), ())),
        preferred_element_type=jnp.float32)
    slot1 = jnp.sum(jnp.where(sel1, start + rankb, 0.0), axis=-1, keepdims=True)
    slot2 = jnp.sum(jnp.where(sel2, start + rankb, 0.0), axis=-1, keepdims=True)
    ss_ref[...] = jnp.where(lane % 2 == 0, slot1, slot2).astype(jnp.int32)

    g_le = (erow <= ecol).astype(jnp.float32)
    cum_tiles = jax.lax.dot_general(
        tiles_per.astype(jnp.float32), g_le, (((1,), (0,)), ((), ())),
        preferred_element_type=jnp.float32).astype(jnp.int32)
    used = jnp.sum(tiles_per, axis=-1, keepdims=True)
    tt = jax.lax.broadcasted_iota(jnp.int32, (1, 128), 1)
    te = jnp.zeros((1, 128), jnp.int32)
    for e in range(E):
        te = te + (tt >= cum_tiles[:, e:e + 1]).astype(jnp.int32)
    te = jnp.minimum(te, E - 1)
    meta_row = jnp.where(tt == NT, used, te)
    meta_ref[...] = jnp.broadcast_to(meta_row, (8, 128))


def _router(x, Wr):
    ss, ww, meta = pl.pallas_call(
        _router_body,
        out_shape=(jax.ShapeDtypeStruct((N, E), jnp.int32),
                   jax.ShapeDtypeStruct((N, E), jnp.float32),
                   jax.ShapeDtypeStruct((8, 128), jnp.int32)),
    )(x, Wr)
    return ss[:, 0], ss[:, 1], ww[:, 0], ww[:, 1], meta


def _mlp_body(meta_ref, xs_ref, wg_ref, wu_ref, wd_ref, os_ref):
    t = pl.program_id(0)

    @pl.when(t < meta_ref[0, NT])
    def _():
        xv = xs_ref[...].astype(jnp.bfloat16)
        g = jax.lax.dot_general(
            xv, wg_ref[0].astype(jnp.bfloat16), (((1,), (1,)), ((), ())),
            preferred_element_type=jnp.float32)
        u = jax.lax.dot_general(
            xv, wu_ref[0].astype(jnp.bfloat16), (((1,), (1,)), ((), ())),
            preferred_element_type=jnp.float32)
        a = g * jax.nn.sigmoid(g) * u
        os_ref[...] = jax.lax.dot_general(
            a.astype(jnp.bfloat16), wd_ref[0].astype(jnp.bfloat16),
            (((1,), (1,)), ((), ())),
            preferred_element_type=jnp.float32)


def _grouped_mlp(meta, xs, Wg, Wu, Wd):
    grid_spec = pltpu.PrefetchScalarGridSpec(
        num_scalar_prefetch=1,
        grid=(NT,),
        in_specs=[
            pl.BlockSpec((T, H), lambda t, m: (t, 0)),
            pl.BlockSpec((1, F, H), lambda t, m: (m[0, t], 0, 0)),
            pl.BlockSpec((1, F, H), lambda t, m: (m[0, t], 0, 0)),
            pl.BlockSpec((1, H, F), lambda t, m: (m[0, t], 0, 0)),
        ],
        out_specs=pl.BlockSpec((T, H), lambda t, m: (t, 0)),
    )
    return pl.pallas_call(
        _mlp_body,
        grid_spec=grid_spec,
        out_shape=jax.ShapeDtypeStruct((S, H), jnp.float32),
        compiler_params=pltpu.CompilerParams(
            dimension_semantics=("arbitrary",)),
    )(meta, xs, Wg, Wu, Wd)


def _sc_mesh():
    return plsc.VectorSubcoreMesh(core_axis_name="c", subcore_axis_name="s")


_NW = 32
_TPW = N // _NW


def _dispatch_body(x_hbm, s1_hbm, s2_hbm, xs_hbm, rows_v, i1_v, i2_v, sem):
    wid = lax.axis_index("s") * 2 + lax.axis_index("c")
    tok0 = wid * _TPW
    pltpu.sync_copy(s1_hbm.at[pl.ds(tok0, _TPW)], i1_v)
    pltpu.sync_copy(s2_hbm.at[pl.ds(tok0, _TPW)], i2_v)
    pltpu.sync_copy(x_hbm.at[pl.ds(tok0, _TPW)], rows_v)
    c1 = pltpu.async_copy(rows_v, xs_hbm.at[i1_v], sem)
    c2 = pltpu.async_copy(rows_v, xs_hbm.at[i2_v], sem)
    c1.wait()
    c2.wait()


def _sc_dispatch(x, slot1, slot2):
    return pl.kernel(
        _dispatch_body,
        mesh=_sc_mesh(),
        out_type=jax.ShapeDtypeStruct((S, H), jnp.float32),
        scratch_types=[
            pltpu.VMEM((_TPW, H), jnp.float32),
            pltpu.VMEM((_TPW,), jnp.int32),
            pltpu.VMEM((_TPW,), jnp.int32),
            pltpu.SemaphoreType.DMA,
        ],
        compiler_params=pltpu.CompilerParams(needs_layout_passes=False),
    )(x, slot1, slot2)


_CSUB = 32


def _combine_body(os_hbm, s1_hbm, s2_hbm, w1_hbm, w2_hbm, out_hbm,
                  r1_v, r2_v, i1_v, i2_v, w1_v, w2_v, sem):
    wid = lax.axis_index("s") * 2 + lax.axis_index("c")
    for sub in range(_TPW // _CSUB):
        tok0 = wid * _TPW + sub * _CSUB
        pltpu.sync_copy(s1_hbm.at[pl.ds(tok0, _CSUB)], i1_v)
        pltpu.sync_copy(s2_hbm.at[pl.ds(tok0, _CSUB)], i2_v)
        pltpu.sync_copy(w1_hbm.at[pl.ds(tok0, _CSUB)], w1_v)
        pltpu.sync_copy(w2_hbm.at[pl.ds(tok0, _CSUB)], w2_v)
        c1 = pltpu.async_copy(os_hbm.at[i1_v], r1_v, sem)
        c2 = pltpu.async_copy(os_hbm.at[i2_v], r2_v, sem)
        c1.wait()
        c2.wait()

        def tok_body(i, _):
            w1b = plsc.load_gather(w1_v, [jnp.full((16,), i, jnp.int32)])
            w2b = plsc.load_gather(w2_v, [jnp.full((16,), i, jnp.int32)])
            for j in range(H // 16):
                sl = pl.ds(j * 16, 16)
                r1_v[i, sl] = w1b * r1_v[i, sl] + w2b * r2_v[i, sl]
            return 0

        lax.fori_loop(0, _CSUB, tok_body, 0)
        pltpu.sync_copy(r1_v, out_hbm.at[pl.ds(tok0, _CSUB)])


def _sc_combine(os_arr, slot1, slot2, w1, w2):
    return pl.kernel(
        _combine_body,
        mesh=_sc_mesh(),
        out_type=jax.ShapeDtypeStruct((N, H), jnp.float32),
        scratch_types=[
            pltpu.VMEM((_CSUB, H), jnp.float32),
            pltpu.VMEM((_CSUB, H), jnp.float32),
            pltpu.VMEM((_CSUB,), jnp.int32),
            pltpu.VMEM((_CSUB,), jnp.int32),
            pltpu.VMEM((_CSUB,), jnp.float32),
            pltpu.VMEM((_CSUB,), jnp.float32),
            pltpu.SemaphoreType.DMA,
        ],
        compiler_params=pltpu.CompilerParams(needs_layout_passes=False),
    )(os_arr, slot1, slot2, w1, w2)


def kernel(x, Wr, Wg, Wu, Wd):
    orig_shape = x.shape
    xf = x.reshape(-1, H)
    slot1, slot2, w1, w2, meta = _router(xf, Wr)

    xs = _sc_dispatch(xf, slot1, slot2)

    os_arr = _grouped_mlp(meta, xs, Wg, Wu, Wd)

    final = _sc_combine(os_arr, slot1, slot2, w1, w2)
    return final.reshape(orig_shape)
